# baseline (speedup 1.0000x reference)
import functools

import jax
import jax.numpy as jnp
from jax import lax
from jax.experimental import pallas as pl
from jax.experimental.pallas import tpu as pltpu
from jax.experimental.pallas import tpu_sc as plsc

N = 10000
NP = 10240
E = 160000
DF = 256
DE = 128

NC = 2
NS = 16
EPT = E // (NC * NS)
CHUNK = 128
NFULL = EPT // CHUNK
TAIL = EPT - NFULL * CHUNK
RPT = NP // NS

NPAIR = (NFULL - 1) // 2


def _sc_mesh():
    return plsc.VectorSubcoreMesh(core_axis_name="c", subcore_axis_name="s")


def _agg_pass(c, s, gs_ref, gd_ref, tab_ref, agg_sp,
              idx_s0, idx_d0, idx_s1, idx_d1, idx_s8, idx_d8,
              rows0, rows1, rows8, sem0, sem1):
    eb = c * (E // NC) + s * EPT

    def load_idx(i, idx_s, idx_d):
        b = pl.multiple_of(eb + i * CHUNK, 8)
        pltpu.sync_copy(gs_ref.at[pl.ds(b, CHUNK)], idx_s)
        pltpu.sync_copy(gd_ref.at[pl.ds(b, CHUNK)], idx_d)

    load_idx(0, idx_s0, idx_d0)
    pltpu.async_copy(tab_ref.at[idx_s0], rows0, sem0)

    def pair(k, carry):
        load_idx(2 * k + 1, idx_s1, idx_d1)
        pltpu.async_copy(tab_ref.at[idx_s1], rows1, sem1)
        pltpu.make_async_copy(tab_ref.at[idx_s0], rows0, sem0).wait()
        pltpu.sync_copy(rows0, agg_sp.at[idx_d0], add=True)
        load_idx(2 * k + 2, idx_s0, idx_d0)
        pltpu.async_copy(tab_ref.at[idx_s0], rows0, sem0)
        pltpu.make_async_copy(tab_ref.at[idx_s1], rows1, sem1).wait()
        pltpu.sync_copy(rows1, agg_sp.at[idx_d1], add=True)
        return carry

    lax.fori_loop(0, NPAIR, pair, 0)
    pltpu.make_async_copy(tab_ref.at[idx_s0], rows0, sem0).wait()
    pltpu.sync_copy(rows0, agg_sp.at[idx_d0], add=True)
    b = pl.multiple_of(eb + NFULL * CHUNK, 8)
    pltpu.sync_copy(gs_ref.at[pl.ds(b, TAIL)], idx_s8)
    pltpu.sync_copy(gd_ref.at[pl.ds(b, TAIL)], idx_d8)
    pltpu.async_copy(tab_ref.at[idx_s8], rows8, sem0).wait()
    pltpu.sync_copy(rows8, agg_sp.at[idx_d8], add=True)


def _sc_aggregate_layer1(xa, xb, gs, gd, z128, ones):
    @functools.partial(
        pl.kernel,
        out_type=[
            jax.ShapeDtypeStruct((NC, 2, NP, 128), jnp.float32),
            jax.ShapeDtypeStruct((NC, NP, 128), jnp.float32),
        ],
        mesh=_sc_mesh(),
        scratch_types=(
            [pltpu.VMEM((CHUNK,), jnp.int32)] * 4
            + [pltpu.VMEM((TAIL,), jnp.int32)] * 2
            + [pltpu.VMEM((CHUNK, 128), jnp.float32)] * 2
            + [pltpu.VMEM((TAIL, 128), jnp.float32)]
            + [pltpu.VMEM_SHARED((NP, 128), jnp.float32)]
            + [pltpu.SemaphoreType.DMA] * 2
        ),
    )
    def k(xa_ref, xb_ref, gs_ref, gd_ref, z128_ref, ones_ref,
          agg_out, deg_out,
          idx_s0, idx_d0, idx_s1, idx_d1, idx_s8, idx_d8,
          rows0, rows1, rows8, agg_sp, sem0, sem1):
        c = lax.axis_index("c")
        s = lax.axis_index("s")
        rb = s * RPT
        eb = c * (E // NC) + s * EPT
        for p in range(3):
            pltpu.sync_copy(z128_ref.at[pl.ds(rb, RPT)], agg_sp.at[pl.ds(rb, RPT)])
            plsc.subcore_barrier()
            if p < 2:
                tab = xa_ref if p == 0 else xb_ref
                _agg_pass(c, s, gs_ref, gd_ref, tab, agg_sp,
                          idx_s0, idx_d0, idx_s1, idx_d1, idx_s8, idx_d8,
                          rows0, rows1, rows8, sem0, sem1)
            else:
                pltpu.sync_copy(ones_ref, rows0)
                pltpu.sync_copy(ones_ref.at[pl.ds(0, TAIL)], rows8)

                def body(i, carry):
                    b = pl.multiple_of(eb + 2 * i * CHUNK, 8)
                    b2 = pl.multiple_of(eb + (2 * i + 1) * CHUNK, 8)
                    pltpu.sync_copy(gd_ref.at[pl.ds(b, CHUNK)], idx_d0)
                    pltpu.sync_copy(gd_ref.at[pl.ds(b2, CHUNK)], idx_d1)
                    pltpu.sync_copy(rows0, agg_sp.at[idx_d0], add=True)
                    pltpu.sync_copy(rows0, agg_sp.at[idx_d1], add=True)
                    return carry

                lax.fori_loop(0, NFULL // 2, body, 0)
                b = pl.multiple_of(eb + (NFULL - 1) * CHUNK, 8)
                pltpu.sync_copy(gd_ref.at[pl.ds(b, CHUNK)], idx_d0)
                pltpu.sync_copy(rows0, agg_sp.at[idx_d0], add=True)
                b = pl.multiple_of(eb + NFULL * CHUNK, 8)
                pltpu.sync_copy(gd_ref.at[pl.ds(b, TAIL)], idx_d8)
                pltpu.sync_copy(rows8, agg_sp.at[idx_d8], add=True)
            plsc.subcore_barrier()
            if p < 2:
                pltpu.sync_copy(agg_sp.at[pl.ds(rb, RPT)],
                                agg_out.at[c, p, pl.ds(rb, RPT)])
            else:
                pltpu.sync_copy(agg_sp.at[pl.ds(rb, RPT)],
                                deg_out.at[c, pl.ds(rb, RPT)])

    return k(xa, xb, gs, gd, z128, ones)


def _sc_aggregate_layer2(y2, gs, gd, z128):
    @functools.partial(
        pl.kernel,
        out_type=[jax.ShapeDtypeStruct((NC, NP, 128), jnp.float32)],
        mesh=_sc_mesh(),
        scratch_types=(
            [pltpu.VMEM((CHUNK,), jnp.int32)] * 4
            + [pltpu.VMEM((TAIL,), jnp.int32)] * 2
            + [pltpu.VMEM((CHUNK, 128), jnp.float32)] * 2
            + [pltpu.VMEM((TAIL, 128), jnp.float32)]
            + [pltpu.VMEM_SHARED((NP, 128), jnp.float32)]
            + [pltpu.SemaphoreType.DMA] * 2
        ),
    )
    def k(y2_ref, gs_ref, gd_ref, z128_ref, agg_out,
          idx_s0, idx_d0, idx_s1, idx_d1, idx_s8, idx_d8,
          rows0, rows1, rows8, agg_sp, sem0, sem1):
        c = lax.axis_index("c")
        s = lax.axis_index("s")
        rb = s * RPT
        pltpu.sync_copy(z128_ref.at[pl.ds(rb, RPT)], agg_sp.at[pl.ds(rb, RPT)])
        plsc.subcore_barrier()
        _agg_pass(c, s, gs_ref, gd_ref, y2_ref, agg_sp,
                  idx_s0, idx_d0, idx_s1, idx_d1, idx_s8, idx_d8,
                  rows0, rows1, rows8, sem0, sem1)
        plsc.subcore_barrier()
        pltpu.sync_copy(agg_sp.at[pl.ds(rb, RPT)],
                        agg_out.at[c, pl.ds(rb, RPT)])

    (agg,) = k(y2, gs, gd, z128)
    return agg


def _tc_layer12(x, a00, a10, a01, a11, d0, d1, Ws1, Wn1, b1, Ws2, Wn2, b2):
    BM = 400
    grid = (N // BM,)
    f32 = jnp.float32

    def body(x_r, a00_r, a10_r, a01_r, a11_r, d0_r, d1_r,
             ws1_r, wn1_r, b1_r, ws2_r, wn2_r, b2_r, y2_r, s2_r):
        deg = d0_r[...] + d1_r[...]
        inv = 1.0 / jnp.maximum(deg[:, 0:1], 1.0)
        hn_l = (a00_r[...] + a10_r[...]) * inv
        hn_r = (a01_r[...] + a11_r[...]) * inv
        h = jnp.dot(x_r[...], ws1_r[...], preferred_element_type=f32)
        h = h + jnp.dot(hn_l, wn1_r[0:128, :], preferred_element_type=f32)
        h = h + jnp.dot(hn_r, wn1_r[128:256, :], preferred_element_type=f32)
        h = jnp.tanh(h + b1_r[...])
        y2_r[...] = jnp.dot(h, wn2_r[...], preferred_element_type=f32)
        s2_r[...] = jnp.dot(h, ws2_r[...], preferred_element_type=f32) + b2_r[...]

    row = lambda i: (i, 0)
    fixed = lambda i: (0, 0)
    return pl.pallas_call(
        body,
        grid=grid,
        in_specs=[
            pl.BlockSpec((BM, DF), row),
            pl.BlockSpec((BM, 128), row),
            pl.BlockSpec((BM, 128), row),
            pl.BlockSpec((BM, 128), row),
            pl.BlockSpec((BM, 128), row),
            pl.BlockSpec((BM, 128), row),
            pl.BlockSpec((BM, 128), row),
            pl.BlockSpec((DF, DF), fixed),
            pl.BlockSpec((DF, DF), fixed),
            pl.BlockSpec((1, DF), fixed),
            pl.BlockSpec((DF, DE), fixed),
            pl.BlockSpec((DF, DE), fixed),
            pl.BlockSpec((1, DE), fixed),
        ],
        out_specs=[
            pl.BlockSpec((BM, DE), row),
            pl.BlockSpec((BM, DE), row),
        ],
        out_shape=[
            jax.ShapeDtypeStruct((N, DE), f32),
            jax.ShapeDtypeStruct((N, DE), f32),
        ],
    )(x, a00, a10, a01, a11, d0, d1, Ws1, Wn1, b1, Ws2, Wn2, b2)


def _tc_final(s2, q0, q1, d0, d1):
    BM = 1000
    grid = (N // BM,)

    def body(s2_r, q0_r, q1_r, d0_r, d1_r, o_r):
        deg = d0_r[...] + d1_r[...]
        inv = 1.0 / jnp.maximum(deg[:, 0:1], 1.0)
        o_r[...] = s2_r[...] + (q0_r[...] + q1_r[...]) * inv

    row = lambda i: (i, 0)
    return pl.pallas_call(
        body,
        grid=grid,
        in_specs=[
            pl.BlockSpec((BM, DE), row),
            pl.BlockSpec((BM, DE), row),
            pl.BlockSpec((BM, DE), row),
            pl.BlockSpec((BM, 128), row),
            pl.BlockSpec((BM, 128), row),
        ],
        out_specs=pl.BlockSpec((BM, DE), row),
        out_shape=jax.ShapeDtypeStruct((N, DE), jnp.float32),
    )(s2, q0, q1, d0, d1)


def kernel(g, inputs, W_self1, W_neigh1, b1, W_self2, W_neigh2, b2):
    f32 = jnp.float32
    xa = inputs[:, :128]
    xb = inputs[:, 128:]
    gs = g[0]
    gd = g[1]
    z128 = jnp.zeros((NP, 128), f32)
    ones = jnp.ones((CHUNK, 128), f32)

    agg, deg = _sc_aggregate_layer1(xa, xb, gs, gd, z128, ones)
    y2, s2 = _tc_layer12(
        inputs, agg[0, 0, :N], agg[1, 0, :N], agg[0, 1, :N], agg[1, 1, :N],
        deg[0, :N], deg[1, :N],
        W_self1, W_neigh1, b1.reshape(1, DF),
        W_self2, W_neigh2, b2.reshape(1, DE),
    )
    agg2 = _sc_aggregate_layer2(y2, gs, gd, z128)
    return _tc_final(s2, agg2[0, :N], agg2[1, :N], deg[0, :N], deg[1, :N])


# speedup vs baseline: 1.0724x; 1.0724x over previous
import functools

import jax
import jax.numpy as jnp
from jax import lax
from jax.experimental import pallas as pl
from jax.experimental.pallas import tpu as pltpu
from jax.experimental.pallas import tpu_sc as plsc

N = 10000
NP = 10240
E = 160000
DF = 256
DE = 128

NC = 2
NS = 16
EPT = E // (NC * NS)
CHUNK = 128
NFULL = EPT // CHUNK
TAIL = EPT - NFULL * CHUNK
RPT = NP // NS

NPAIR = (NFULL - 1) // 2


def _sc_mesh():
    return plsc.VectorSubcoreMesh(core_axis_name="c", subcore_axis_name="s")


def _agg_pass(c, s, gs_ref, gd_ref, tab_ref, agg_sp,
              idx_s0, idx_d0, idx_s1, idx_d1, idx_s8, idx_d8,
              rows0, rows1, rows8, sem0, sem1):
    eb = c * (E // NC) + s * EPT

    def load_idx(i, idx_s, idx_d):
        b = pl.multiple_of(eb + i * CHUNK, 8)
        pltpu.sync_copy(gs_ref.at[pl.ds(b, CHUNK)], idx_s)
        pltpu.sync_copy(gd_ref.at[pl.ds(b, CHUNK)], idx_d)

    load_idx(0, idx_s0, idx_d0)
    pltpu.async_copy(tab_ref.at[idx_s0], rows0, sem0)

    def pair(k, carry):
        load_idx(2 * k + 1, idx_s1, idx_d1)
        pltpu.async_copy(tab_ref.at[idx_s1], rows1, sem1)
        pltpu.make_async_copy(tab_ref.at[idx_s0], rows0, sem0).wait()
        pltpu.sync_copy(rows0, agg_sp.at[idx_d0], add=True)
        load_idx(2 * k + 2, idx_s0, idx_d0)
        pltpu.async_copy(tab_ref.at[idx_s0], rows0, sem0)
        pltpu.make_async_copy(tab_ref.at[idx_s1], rows1, sem1).wait()
        pltpu.sync_copy(rows1, agg_sp.at[idx_d1], add=True)
        return carry

    lax.fori_loop(0, NPAIR, pair, 0)
    pltpu.make_async_copy(tab_ref.at[idx_s0], rows0, sem0).wait()
    pltpu.sync_copy(rows0, agg_sp.at[idx_d0], add=True)
    b = pl.multiple_of(eb + NFULL * CHUNK, 8)
    pltpu.sync_copy(gs_ref.at[pl.ds(b, TAIL)], idx_s8)
    pltpu.sync_copy(gd_ref.at[pl.ds(b, TAIL)], idx_d8)
    pltpu.async_copy(tab_ref.at[idx_s8], rows8, sem0).wait()
    pltpu.sync_copy(rows8, agg_sp.at[idx_d8], add=True)


def _sc_aggregate_layer1(xa, xb, gs, gd, z128, ones):
    @functools.partial(
        pl.kernel,
        out_type=[
            jax.ShapeDtypeStruct((NC, 2, NP, 128), jnp.float32),
            jax.ShapeDtypeStruct((NC, NP, 128), jnp.float32),
        ],
        mesh=_sc_mesh(),
        scratch_types=(
            [pltpu.VMEM((CHUNK,), jnp.int32)] * 4
            + [pltpu.VMEM((TAIL,), jnp.int32)] * 2
            + [pltpu.VMEM((CHUNK, 128), jnp.float32)] * 2
            + [pltpu.VMEM((TAIL, 128), jnp.float32)]
            + [pltpu.VMEM_SHARED((NP, 128), jnp.float32)]
            + [pltpu.SemaphoreType.DMA] * 2
        ),
    )
    def k(xa_ref, xb_ref, gs_ref, gd_ref, z128_ref, ones_ref,
          agg_out, deg_out,
          idx_s0, idx_d0, idx_s1, idx_d1, idx_s8, idx_d8,
          rows0, rows1, rows8, agg_sp, sem0, sem1):
        c = lax.axis_index("c")
        s = lax.axis_index("s")
        rb = s * RPT
        eb = c * (E // NC) + s * EPT
        for p in range(3):
            pltpu.sync_copy(z128_ref.at[pl.ds(rb, RPT)], agg_sp.at[pl.ds(rb, RPT)])
            plsc.subcore_barrier()
            if p < 2:
                tab = xa_ref if p == 0 else xb_ref
                _agg_pass(c, s, gs_ref, gd_ref, tab, agg_sp,
                          idx_s0, idx_d0, idx_s1, idx_d1, idx_s8, idx_d8,
                          rows0, rows1, rows8, sem0, sem1)
            else:
                pltpu.sync_copy(ones_ref, rows0)
                pltpu.sync_copy(ones_ref.at[pl.ds(0, TAIL)], rows8)

                def body(i, carry):
                    b = pl.multiple_of(eb + 2 * i * CHUNK, 8)
                    b2 = pl.multiple_of(eb + (2 * i + 1) * CHUNK, 8)
                    pltpu.sync_copy(gd_ref.at[pl.ds(b, CHUNK)], idx_d0)
                    pltpu.sync_copy(gd_ref.at[pl.ds(b2, CHUNK)], idx_d1)
                    pltpu.sync_copy(rows0, agg_sp.at[idx_d0], add=True)
                    pltpu.sync_copy(rows0, agg_sp.at[idx_d1], add=True)
                    return carry

                lax.fori_loop(0, NFULL // 2, body, 0)
                b = pl.multiple_of(eb + (NFULL - 1) * CHUNK, 8)
                pltpu.sync_copy(gd_ref.at[pl.ds(b, CHUNK)], idx_d0)
                pltpu.sync_copy(rows0, agg_sp.at[idx_d0], add=True)
                b = pl.multiple_of(eb + NFULL * CHUNK, 8)
                pltpu.sync_copy(gd_ref.at[pl.ds(b, TAIL)], idx_d8)
                pltpu.sync_copy(rows8, agg_sp.at[idx_d8], add=True)
            plsc.subcore_barrier()
            if p < 2:
                pltpu.sync_copy(agg_sp.at[pl.ds(rb, RPT)],
                                agg_out.at[c, p, pl.ds(rb, RPT)])
            else:
                pltpu.sync_copy(agg_sp.at[pl.ds(rb, RPT)],
                                deg_out.at[c, pl.ds(rb, RPT)])

    return k(xa, xb, gs, gd, z128, ones)


def _sc_aggregate_layer2(y2, gs, gd, z128):
    @functools.partial(
        pl.kernel,
        out_type=[jax.ShapeDtypeStruct((NC, NP, 128), jnp.float32)],
        mesh=_sc_mesh(),
        scratch_types=(
            [pltpu.VMEM((CHUNK,), jnp.int32)] * 4
            + [pltpu.VMEM((TAIL,), jnp.int32)] * 2
            + [pltpu.VMEM((CHUNK, 128), jnp.float32)] * 2
            + [pltpu.VMEM((TAIL, 128), jnp.float32)]
            + [pltpu.VMEM_SHARED((NP, 128), jnp.float32)]
            + [pltpu.SemaphoreType.DMA] * 2
        ),
    )
    def k(y2_ref, gs_ref, gd_ref, z128_ref, agg_out,
          idx_s0, idx_d0, idx_s1, idx_d1, idx_s8, idx_d8,
          rows0, rows1, rows8, agg_sp, sem0, sem1):
        c = lax.axis_index("c")
        s = lax.axis_index("s")
        rb = s * RPT
        pltpu.sync_copy(z128_ref.at[pl.ds(rb, RPT)], agg_sp.at[pl.ds(rb, RPT)])
        plsc.subcore_barrier()
        _agg_pass(c, s, gs_ref, gd_ref, y2_ref, agg_sp,
                  idx_s0, idx_d0, idx_s1, idx_d1, idx_s8, idx_d8,
                  rows0, rows1, rows8, sem0, sem1)
        plsc.subcore_barrier()
        pltpu.sync_copy(agg_sp.at[pl.ds(rb, RPT)],
                        agg_out.at[c, pl.ds(rb, RPT)])

    (agg,) = k(y2, gs, gd, z128)
    return agg


def _tc_layer12(x, agg, deg, Ws1, Wn1, b1, Ws2, Wn2, b2):
    BM = 400
    grid = (N // BM,)
    f32 = jnp.float32

    def body(x_r, a00_r, a10_r, a01_r, a11_r, d0_r, d1_r,
             ws1_r, wn1_r, b1_r, ws2_r, wn2_r, b2_r, y2_r, s2_r):
        deg_b = (d0_r[...] + d1_r[...]).reshape(BM, 128)
        inv = 1.0 / jnp.maximum(deg_b[:, 0:1], 1.0)
        hn_l = (a00_r[...] + a10_r[...]).reshape(BM, 128) * inv
        hn_r = (a01_r[...] + a11_r[...]).reshape(BM, 128) * inv
        h = jnp.dot(x_r[...], ws1_r[...], preferred_element_type=f32)
        h = h + jnp.dot(hn_l, wn1_r[0:128, :], preferred_element_type=f32)
        h = h + jnp.dot(hn_r, wn1_r[128:256, :], preferred_element_type=f32)
        h = jnp.tanh(h + b1_r[...])
        y2_r[...] = jnp.dot(h, wn2_r[...], preferred_element_type=f32)
        s2_r[...] = jnp.dot(h, ws2_r[...], preferred_element_type=f32) + b2_r[...]

    row = lambda i: (i, 0)
    fixed = lambda i: (0, 0)
    agg_spec = lambda cc, pp: pl.BlockSpec((1, 1, BM, 128),
                                           lambda i: (cc, pp, i, 0))
    deg_spec = lambda cc: pl.BlockSpec((1, BM, 128), lambda i: (cc, i, 0))
    return pl.pallas_call(
        body,
        grid=grid,
        in_specs=[
            pl.BlockSpec((BM, DF), row),
            agg_spec(0, 0),
            agg_spec(1, 0),
            agg_spec(0, 1),
            agg_spec(1, 1),
            deg_spec(0),
            deg_spec(1),
            pl.BlockSpec((DF, DF), fixed),
            pl.BlockSpec((DF, DF), fixed),
            pl.BlockSpec((1, DF), fixed),
            pl.BlockSpec((DF, DE), fixed),
            pl.BlockSpec((DF, DE), fixed),
            pl.BlockSpec((1, DE), fixed),
        ],
        out_specs=[
            pl.BlockSpec((BM, DE), row),
            pl.BlockSpec((BM, DE), row),
        ],
        out_shape=[
            jax.ShapeDtypeStruct((N, DE), f32),
            jax.ShapeDtypeStruct((N, DE), f32),
        ],
    )(x, agg, agg, agg, agg, deg, deg, Ws1, Wn1, b1, Ws2, Wn2, b2)


def _tc_final(s2, agg2, deg):
    BM = 1000
    grid = (N // BM,)

    def body(s2_r, q0_r, q1_r, d0_r, d1_r, o_r):
        deg_b = (d0_r[...] + d1_r[...]).reshape(BM, 128)
        inv = 1.0 / jnp.maximum(deg_b[:, 0:1], 1.0)
        q = (q0_r[...] + q1_r[...]).reshape(BM, DE)
        o_r[...] = s2_r[...] + q * inv

    row = lambda i: (i, 0)
    part_spec = lambda cc: pl.BlockSpec((1, BM, 128), lambda i: (cc, i, 0))
    return pl.pallas_call(
        body,
        grid=grid,
        in_specs=[
            pl.BlockSpec((BM, DE), row),
            part_spec(0),
            part_spec(1),
            part_spec(0),
            part_spec(1),
        ],
        out_specs=pl.BlockSpec((BM, DE), row),
        out_shape=jax.ShapeDtypeStruct((N, DE), jnp.float32),
    )(s2, agg2, agg2, deg, deg)


def kernel(g, inputs, W_self1, W_neigh1, b1, W_self2, W_neigh2, b2):
    f32 = jnp.float32
    xa = inputs[:, :128]
    xb = inputs[:, 128:]
    gs = g[0]
    gd = g[1]
    z128 = jnp.zeros((NP, 128), f32)
    ones = jnp.ones((CHUNK, 128), f32)

    agg, deg = _sc_aggregate_layer1(xa, xb, gs, gd, z128, ones)
    y2, s2 = _tc_layer12(
        inputs, agg, deg,
        W_self1, W_neigh1, b1.reshape(1, DF),
        W_self2, W_neigh2, b2.reshape(1, DE),
    )
    agg2 = _sc_aggregate_layer2(y2, gs, gd, z128)
    return _tc_final(s2, agg2, deg)
